# baseline (speedup 1.0000x reference)
import functools

import jax
import jax.numpy as jnp
from jax import lax
from jax.experimental import pallas as pl
from jax.experimental.pallas import tpu as pltpu
from jax.experimental.pallas import tpu_sc as plsc

N = 10000
D = 128
H = 128
G = 64
K = 3
NUM_LAYERS = 2

NP = 10240
RB = 512
NCHUNKS = NP // RB
NC = 2
NS = 16
NW = NC * NS
ROWS_PER_TILE = NP // NS
CW = 128

NEG = -1e30
IMAX = 2**31 - 1


def _embed_body(x_ref, wt_ref, b_ref, brow_ref, bcol_ref,
                h_ref, xn_ref, ind_ref, indt_ref):
    h = jnp.dot(x_ref[...], wt_ref[...], preferred_element_type=jnp.float32)
    h = h + b_ref[...]
    h_ref[...] = h
    nrm = jnp.sqrt(jnp.sum(h * h, axis=1, keepdims=True))
    xn_ref[...] = h / jnp.maximum(nrm, 1e-12)
    gid_r = lax.broadcasted_iota(jnp.int32, (G, NP), 0)
    ind_ref[...] = (gid_r == brow_ref[...]).astype(jnp.float32)
    gid_c = lax.broadcasted_iota(jnp.int32, (NP, G), 1)
    indt_ref[...] = (gid_c == bcol_ref[...]).astype(jnp.float32)


def _embed(x_p, wembT, b_emb, brow, bcol):
    return pl.pallas_call(
        _embed_body,
        out_shape=(
            jax.ShapeDtypeStruct((NP, H), jnp.float32),
            jax.ShapeDtypeStruct((NP, H), jnp.float32),
            jax.ShapeDtypeStruct((G, NP), jnp.float32),
            jax.ShapeDtypeStruct((NP, G), jnp.float32),
        ),
    )(x_p, wembT, b_emb, brow, bcol)


def _knn_body(bmin_ref, bmax_ref, xn_ref, brow_ref, bcol_ref, nbr_ref,
              rv_ref, ri_ref):
    i = pl.program_id(0)
    rmin = bmin_ref[i]
    rmax = bmax_ref[i]
    xb = xn_ref[pl.ds(pl.multiple_of(i * RB, RB), RB), :]
    bb = bcol_ref[...]
    rv_ref[...] = jnp.full((RB, K), NEG, jnp.float32)
    ri_ref[...] = jnp.zeros((RB, K), jnp.int32)
    rowid = i * RB + lax.broadcasted_iota(jnp.int32, (RB, RB), 0)

    def chunk(j, _):
        cmin = bmin_ref[j]
        cmax = bmax_ref[j]

        @pl.when(jnp.logical_and(cmin <= rmax, cmax >= rmin))
        def _():
            off = pl.multiple_of(j * RB, RB)
            xc = xn_ref[pl.ds(off, RB), :]
            bc = brow_ref[:, pl.ds(off, RB)]
            sim = lax.dot_general(xb, xc, (((1,), (1,)), ((), ())),
                                  preferred_element_type=jnp.float32)
            colid = j * RB + lax.broadcasted_iota(jnp.int32, (RB, RB), 1)
            valid = jnp.logical_and(bb == bc, colid != rowid)
            s = jnp.where(valid, sim, NEG)
            lv, li = [], []
            for _t in range(K):
                m = jnp.max(s, axis=1, keepdims=True)
                sel = jnp.where(s == m, colid, IMAX)
                jm = jnp.min(sel, axis=1, keepdims=True)
                lv.append(m)
                li.append(jm)
                s = jnp.where(colid == jm, NEG, s)
            cand_v = jnp.concatenate([rv_ref[...]] + lv, axis=1)
            cand_i = jnp.concatenate([ri_ref[...]] + li, axis=1)
            nv, ni = [], []
            for _t in range(K):
                m = jnp.max(cand_v, axis=1, keepdims=True)
                sel = jnp.where(cand_v == m, cand_i, IMAX)
                jm = jnp.min(sel, axis=1, keepdims=True)
                nv.append(m)
                ni.append(jm)
                kill = jnp.logical_and(cand_i == jm, cand_v == m)
                cand_v = jnp.where(kill, NEG, cand_v)
                cand_i = jnp.where(kill, IMAX, cand_i)
            rv_ref[...] = jnp.concatenate(nv, axis=1)
            ri_ref[...] = jnp.concatenate(ni, axis=1)
        return 0

    lax.fori_loop(0, NCHUNKS, chunk, 0)
    nbr_ref[...] = jnp.minimum(ri_ref[...], NP - 1)


def _knn(xn, brow, bcol, bmin, bmax):
    return pl.pallas_call(
        _knn_body,
        grid=(NCHUNKS,),
        in_specs=[
            pl.BlockSpec(memory_space=pltpu.SMEM),
            pl.BlockSpec(memory_space=pltpu.SMEM),
            pl.BlockSpec((NP, H), lambda i: (0, 0)),
            pl.BlockSpec((1, NP), lambda i: (0, 0)),
            pl.BlockSpec((RB, 1), lambda i: (i, 0)),
        ],
        out_specs=pl.BlockSpec((RB, K), lambda i: (i, 0)),
        out_shape=jax.ShapeDtypeStruct((NP, K), jnp.int32),
        scratch_shapes=[
            pltpu.VMEM((RB, K), jnp.float32),
            pltpu.VMEM((RB, K), jnp.int32),
        ],
    )(bmin, bmax, xn, brow, bcol)


def _hist_kernel(nch0, nch1):
    mesh = plsc.VectorSubcoreMesh(core_axis_name="c", subcore_axis_name="s")

    @functools.partial(
        pl.kernel, mesh=mesh,
        out_type=jax.ShapeDtypeStruct((NC, NP), jnp.float32),
        scratch_types=[
            pltpu.VMEM((nch0, CW), jnp.int32),
            pltpu.VMEM((CW,), jnp.float32),
            pltpu.VMEM((ROWS_PER_TILE,), jnp.float32),
            pltpu.VMEM_SHARED((NP,), jnp.float32),
        ],
    )
    def hist(dst_hbm, out_hbm, idx_v, ones_v, zer_v, acc):
        cid = lax.axis_index("c")
        sid = lax.axis_index("s")
        w = cid * NS + sid
        nch_c = jnp.where(cid == 0, nch0, nch1)
        for t in range(CW // 16):
            ones_v[pl.ds(16 * t, 16)] = jnp.ones((16,), jnp.float32)
        for t in range(ROWS_PER_TILE // 16):
            zer_v[pl.ds(16 * t, 16)] = jnp.zeros((16,), jnp.float32)
        pltpu.sync_copy(zer_v, acc.at[pl.ds(sid * ROWS_PER_TILE, ROWS_PER_TILE)])
        plsc.subcore_barrier()
        pltpu.sync_copy(dst_hbm.at[w], idx_v)

        def body(c, _):
            pltpu.sync_copy(ones_v, acc.at[idx_v.at[c]], add=True)
            return 0

        lax.fori_loop(0, nch_c, body, 0)
        plsc.subcore_barrier()
        sl = pl.ds(sid * ROWS_PER_TILE, ROWS_PER_TILE)
        pltpu.sync_copy(acc.at[sl], out_hbm.at[cid].at[sl])

    return hist


def _agg_kernel(nch0, nch1, ph):
    mesh = plsc.VectorSubcoreMesh(core_axis_name="c", subcore_axis_name="s")

    @functools.partial(
        pl.kernel, mesh=mesh,
        out_type=jax.ShapeDtypeStruct((NC, NP, H), jnp.float32),
        scratch_types=[
            pltpu.VMEM((ph, CW), jnp.int32),
            pltpu.VMEM((ph, CW), jnp.int32),
            pltpu.VMEM((CW, H), jnp.float32),
            pltpu.VMEM((CW, H), jnp.float32),
            pltpu.SemaphoreType.DMA,
            pltpu.SemaphoreType.DMA,
            pltpu.VMEM_SHARED((NP, H), jnp.float32),
        ],
    )
    def agg(y_hbm, src_hbm, dsti_hbm, zeros_hbm, out_hbm,
            src_v, dst_v, buf0, buf1, sem0, sem1, acc):
        cid = lax.axis_index("c")
        sid = lax.axis_index("s")
        w = cid * NS + sid
        nph = jnp.where(cid == 0, nch0 // ph, nch1 // ph)
        rsl = pl.ds(sid * ROWS_PER_TILE, ROWS_PER_TILE)
        pltpu.sync_copy(zeros_hbm, acc.at[rsl])
        plsc.subcore_barrier()

        def phase(p, _):
            off = p * ph
            pltpu.sync_copy(src_hbm.at[w].at[pl.ds(off, ph)], src_v)
            pltpu.sync_copy(dsti_hbm.at[w].at[pl.ds(off, ph)], dst_v)
            pltpu.async_copy(y_hbm.at[src_v.at[0]], buf0, sem0)

            def body(i, _):
                c0 = i * 2
                c1 = i * 2 + 1
                pltpu.async_copy(y_hbm.at[src_v.at[c1]], buf1, sem1)
                pltpu.make_async_copy(y_hbm.at[src_v.at[c0]], buf0, sem0).wait()
                pltpu.sync_copy(buf0, acc.at[dst_v.at[c0]], add=True)

                @pl.when(i < ph // 2 - 1)
                def _():
                    pltpu.async_copy(y_hbm.at[src_v.at[c0 + 2]], buf0, sem0)

                pltpu.make_async_copy(y_hbm.at[src_v.at[c1]], buf1, sem1).wait()
                pltpu.sync_copy(buf1, acc.at[dst_v.at[c1]], add=True)
                return 0

            lax.fori_loop(0, ph // 2, body, 0)
            return 0

        lax.fori_loop(0, nph, phase, 0)
        plsc.subcore_barrier()
        pltpu.sync_copy(acc.at[rsl], out_hbm.at[cid].at[rsl])

    return agg


def _graphnorm(X, w, b, ms, ind, indt, cnt):
    mean = jnp.dot(ind, X, preferred_element_type=jnp.float32) / cnt
    cen = X - ms * jnp.dot(indt, mean, preferred_element_type=jnp.float32)
    var = jnp.dot(ind, cen * cen, preferred_element_type=jnp.float32) / cnt
    invstd = lax.rsqrt(var + 1e-5)
    return w * cen * jnp.dot(indt, invstd, preferred_element_type=jnp.float32) + b


def _leaky(v):
    return jnp.where(v >= 0, v, 0.01 * v)


def _prep_body(h_ref, wt_ref, h0_ref, h1_ref, y_ref, dinv_ref):
    dinv = lax.rsqrt(1.0 + h0_ref[...] + h1_ref[...])
    dinv_ref[...] = dinv
    y_ref[...] = jnp.dot(h_ref[...], wt_ref[...],
                         preferred_element_type=jnp.float32) * dinv


def _prep(h, wt, hist0, hist1):
    return pl.pallas_call(
        _prep_body,
        out_shape=(jax.ShapeDtypeStruct((NP, H), jnp.float32),
                   jax.ShapeDtypeStruct((NP, 1), jnp.float32)),
    )(h, wt, hist0, hist1)


def _road_post_body(p0_ref, p1_ref, y_ref, dinv_ref, cb_ref,
                    nw_ref, nb_ref, nms_ref, ind_ref, indt_ref, fwt_ref,
                    h_ref, y2_ref):
    ind = ind_ref[...]
    indt = indt_ref[...]
    cnt = jnp.maximum(jnp.sum(ind, axis=1, keepdims=True), 1.0)
    X = dinv_ref[...] * (p0_ref[...] + p1_ref[...] + y_ref[...]) + cb_ref[...]
    X = _leaky(_graphnorm(X, nw_ref[...], nb_ref[...], nms_ref[...],
                          ind, indt, cnt))
    h_ref[...] = X
    y2_ref[...] = jnp.dot(X, fwt_ref[...], preferred_element_type=jnp.float32)


def _road_post(p0, p1, y, dinv, cb, nw, nb, nms, ind, indt, fwt):
    return pl.pallas_call(
        _road_post_body,
        out_shape=(jax.ShapeDtypeStruct((NP, H), jnp.float32),
                   jax.ShapeDtypeStruct((NP, H), jnp.float32)),
    )(p0, p1, y, dinv, cb, nw, nb, nms, ind, indt, fwt)


def _feat_post_body(q0_ref, q1_ref, y2_ref, hp_ref, fcb_ref,
                    nw_ref, nb_ref, nms_ref, ind_ref, indt_ref,
                    cwt_ref, dinv_ref, h_ref, y_ref):
    ind = ind_ref[...]
    indt = indt_ref[...]
    cnt = jnp.maximum(jnp.sum(ind, axis=1, keepdims=True), 1.0)
    F = 0.25 * (q0_ref[...] + q1_ref[...] + y2_ref[...]) + fcb_ref[...]
    F = _leaky(_graphnorm(F, nw_ref[...], nb_ref[...], nms_ref[...],
                          ind, indt, cnt))
    hn = (hp_ref[...] + F) / 2.0
    h_ref[...] = hn
    y_ref[...] = jnp.dot(hn, cwt_ref[...],
                         preferred_element_type=jnp.float32) * dinv_ref[...]


def _feat_post(q0, q1, y2, hp, fcb, nw, nb, nms, ind, indt, cwt, dinv):
    return pl.pallas_call(
        _feat_post_body,
        out_shape=(jax.ShapeDtypeStruct((NP, H), jnp.float32),
                   jax.ShapeDtypeStruct((NP, H), jnp.float32)),
    )(q0, q1, y2, hp, fcb, nw, nb, nms, ind, indt, cwt, dinv)


def _final_body(q0_ref, q1_ref, y2_ref, hp_ref, fcb_ref,
                nw_ref, nb_ref, nms_ref, ind_ref, indt_ref, h1_ref, gf_ref):
    ind = ind_ref[...]
    indt = indt_ref[...]
    cnt = jnp.maximum(jnp.sum(ind, axis=1, keepdims=True), 1.0)
    F = 0.25 * (q0_ref[...] + q1_ref[...] + y2_ref[...]) + fcb_ref[...]
    F = _leaky(_graphnorm(F, nw_ref[...], nb_ref[...], nms_ref[...],
                          ind, indt, cnt))
    h2 = (hp_ref[...] + F) / 2.0
    p1 = jnp.dot(ind, h1_ref[...], preferred_element_type=jnp.float32) / cnt
    p2 = jnp.dot(ind, h2, preferred_element_type=jnp.float32) / cnt
    gf_ref[...] = p1 + 2.0 * p2


def _final(q0, q1, y2, hp, fcb, nw, nb, nms, ind, indt, h1):
    return pl.pallas_call(
        _final_body,
        out_shape=jax.ShapeDtypeStruct((G, H), jnp.float32),
    )(q0, q1, y2, hp, fcb, nw, nb, nms, ind, indt, h1)


def _pad_edges(src, dst, nch0, nch1):
    cap0 = NS * nch0 * CW
    tot = cap0 + NS * nch1 * CW
    pad = tot - src.shape[0]

    def build(a):
        a_p = jnp.concatenate([a, jnp.full((pad,), NP - 1, jnp.int32)])
        a0 = a_p[:cap0].reshape(NS, nch0, CW)
        a1 = a_p[cap0:].reshape(NS, nch1, CW)
        a1 = jnp.concatenate(
            [a1, jnp.full((NS, nch0 - nch1, CW), NP - 1, jnp.int32)], axis=1)
        return jnp.concatenate([a0, a1], axis=0)

    return build(src), build(dst)


def kernel(x, edge_index, batch, W_emb, b_emb, conv_W, conv_b, norm_w, norm_b,
           norm_ms, fconv_W, fconv_b, fnorm_w, fnorm_b, fnorm_ms):
    pad = NP - N
    x_p = jnp.concatenate([x, jnp.zeros((pad, D), x.dtype)])
    batch_p = jnp.concatenate([batch, jnp.full((pad,), G + 63, jnp.int32)])
    brow = batch_p.reshape(1, NP)
    bcol = batch_p.reshape(NP, 1)
    bmin = batch_p.reshape(NCHUNKS, RB).min(axis=1)
    bmax = batch_p.reshape(NCHUNKS, RB).max(axis=1)

    h0, xn, ind, indt = _embed(x_p, W_emb.T, b_emb.reshape(1, H), brow, bcol)
    nbr = _knn(xn, brow, bcol, bmin, bmax)

    src_p, dst_p = _pad_edges(edge_index[0], edge_index[1], 160, 0)
    fsrc = nbr[:N].reshape(N * K)
    fdst = jnp.repeat(jnp.arange(N, dtype=jnp.int32), K)
    fsrc_p, fdst_p = _pad_edges(fsrc, fdst, 16, 0)

    hist = _hist_kernel(160, 0)(dst_p)
    hist0 = hist[0].reshape(NP, 1)
    hist1 = hist[1].reshape(NP, 1)

    zeros_rows = jnp.zeros((ROWS_PER_TILE, H), jnp.float32)
    agg_r = _agg_kernel(160, 0, 32)
    agg_f = _agg_kernel(16, 0, 4)

    h = h0
    y, dinv = _prep(h, conv_W[0].T, hist0, hist1)
    p = agg_r(y, src_p, dst_p, zeros_rows)
    hp, y2 = _road_post(p[0], p[1], y, dinv, conv_b[0].reshape(1, H),
                        norm_w[0].reshape(1, H), norm_b[0].reshape(1, H),
                        norm_ms[0].reshape(1, H), ind, indt, fconv_W[0].T)
    q = agg_f(y2, fsrc_p, fdst_p, zeros_rows)
    h1, y = _feat_post(q[0], q[1], y2, hp, fconv_b[0].reshape(1, H),
                       fnorm_w[0].reshape(1, H), fnorm_b[0].reshape(1, H),
                       fnorm_ms[0].reshape(1, H), ind, indt,
                       conv_W[1].T, dinv)

    p = agg_r(y, src_p, dst_p, zeros_rows)
    hp, y2 = _road_post(p[0], p[1], y, dinv, conv_b[1].reshape(1, H),
                        norm_w[1].reshape(1, H), norm_b[1].reshape(1, H),
                        norm_ms[1].reshape(1, H), ind, indt, fconv_W[1].T)
    q = agg_f(y2, fsrc_p, fdst_p, zeros_rows)
    gf = _final(q[0], q[1], y2, hp, fconv_b[1].reshape(1, H),
                fnorm_w[1].reshape(1, H), fnorm_b[1].reshape(1, H),
                fnorm_ms[1].reshape(1, H), ind, indt, h1)
    return (gf, 0)


# speedup vs baseline: 1.2902x; 1.2902x over previous
import functools

import jax
import jax.numpy as jnp
from jax import lax
from jax.experimental import pallas as pl
from jax.experimental.pallas import tpu as pltpu
from jax.experimental.pallas import tpu_sc as plsc

N = 10000
D = 128
H = 128
G = 64
K = 3
NUM_LAYERS = 2

NP = 10240
RB = 512
NCHUNKS = NP // RB
NC = 2
NS = 16
NW = NC * NS
ROWS_PER_TILE = NP // NS
CW = 128

NEG = -1e30
IMAX = 2**31 - 1


def _embed_body(x_ref, wt_ref, b_ref, brow_ref, bcol_ref,
                h_ref, xn_ref, ind_ref, indt_ref):
    h = jnp.dot(x_ref[...], wt_ref[...], preferred_element_type=jnp.float32)
    h = h + b_ref[...]
    h_ref[...] = h
    nrm = jnp.sqrt(jnp.sum(h * h, axis=1, keepdims=True))
    xn_ref[...] = h / jnp.maximum(nrm, 1e-12)
    gid_r = lax.broadcasted_iota(jnp.int32, (G, NP), 0)
    ind_ref[...] = (gid_r == brow_ref[...]).astype(jnp.float32)
    gid_c = lax.broadcasted_iota(jnp.int32, (NP, G), 1)
    indt_ref[...] = (gid_c == bcol_ref[...]).astype(jnp.float32)


def _embed(x_p, wembT, b_emb, brow, bcol):
    return pl.pallas_call(
        _embed_body,
        out_shape=(
            jax.ShapeDtypeStruct((NP, H), jnp.float32),
            jax.ShapeDtypeStruct((NP, H), jnp.float32),
            jax.ShapeDtypeStruct((G, NP), jnp.float32),
            jax.ShapeDtypeStruct((NP, G), jnp.float32),
        ),
    )(x_p, wembT, b_emb, brow, bcol)


def _knn_body(bmin_ref, bmax_ref, xn_ref, brow_ref, bcol_ref, nbr_ref,
              rv_ref, ri_ref):
    i = pl.program_id(0)
    rmin = bmin_ref[i]
    rmax = bmax_ref[i]
    xb = xn_ref[pl.ds(pl.multiple_of(i * RB, RB), RB), :]
    bb = bcol_ref[...]
    rv_ref[...] = jnp.full((RB, K), NEG, jnp.float32)
    ri_ref[...] = jnp.zeros((RB, K), jnp.int32)
    rowid = i * RB + lax.broadcasted_iota(jnp.int32, (RB, RB), 0)

    def chunk(j, _):
        cmin = bmin_ref[j]
        cmax = bmax_ref[j]

        @pl.when(jnp.logical_and(cmin <= rmax, cmax >= rmin))
        def _():
            off = pl.multiple_of(j * RB, RB)
            xc = xn_ref[pl.ds(off, RB), :]
            bc = brow_ref[:, pl.ds(off, RB)]
            sim = lax.dot_general(xb, xc, (((1,), (1,)), ((), ())),
                                  preferred_element_type=jnp.float32)
            colid = j * RB + lax.broadcasted_iota(jnp.int32, (RB, RB), 1)
            valid = jnp.logical_and(bb == bc, colid != rowid)
            s = jnp.where(valid, sim, NEG)
            lv, li = [], []
            for _t in range(K):
                m = jnp.max(s, axis=1, keepdims=True)
                sel = jnp.where(s == m, colid, IMAX)
                jm = jnp.min(sel, axis=1, keepdims=True)
                lv.append(m)
                li.append(jm)
                s = jnp.where(colid == jm, NEG, s)
            cand_v = jnp.concatenate([rv_ref[...]] + lv, axis=1)
            cand_i = jnp.concatenate([ri_ref[...]] + li, axis=1)
            nv, ni = [], []
            for _t in range(K):
                m = jnp.max(cand_v, axis=1, keepdims=True)
                sel = jnp.where(cand_v == m, cand_i, IMAX)
                jm = jnp.min(sel, axis=1, keepdims=True)
                nv.append(m)
                ni.append(jm)
                kill = jnp.logical_and(cand_i == jm, cand_v == m)
                cand_v = jnp.where(kill, NEG, cand_v)
                cand_i = jnp.where(kill, IMAX, cand_i)
            rv_ref[...] = jnp.concatenate(nv, axis=1)
            ri_ref[...] = jnp.concatenate(ni, axis=1)
        return 0

    lax.fori_loop(0, NCHUNKS, chunk, 0)
    nbr_ref[...] = jnp.minimum(ri_ref[...], NP - 1)


def _knn(xn, brow, bcol, bmin, bmax):
    return pl.pallas_call(
        _knn_body,
        grid=(NCHUNKS,),
        in_specs=[
            pl.BlockSpec(memory_space=pltpu.SMEM),
            pl.BlockSpec(memory_space=pltpu.SMEM),
            pl.BlockSpec((NP, H), lambda i: (0, 0)),
            pl.BlockSpec((1, NP), lambda i: (0, 0)),
            pl.BlockSpec((RB, 1), lambda i: (i, 0)),
        ],
        out_specs=pl.BlockSpec((RB, K), lambda i: (i, 0)),
        out_shape=jax.ShapeDtypeStruct((NP, K), jnp.int32),
        scratch_shapes=[
            pltpu.VMEM((RB, K), jnp.float32),
            pltpu.VMEM((RB, K), jnp.int32),
        ],
    )(bmin, bmax, xn, brow, bcol)


def _hist_kernel(nch0, nch1):
    mesh = plsc.VectorSubcoreMesh(core_axis_name="c", subcore_axis_name="s")

    @functools.partial(
        pl.kernel, mesh=mesh,
        out_type=jax.ShapeDtypeStruct((NC, NP), jnp.float32),
        scratch_types=[
            pltpu.VMEM((nch0, CW), jnp.int32),
            pltpu.VMEM((CW,), jnp.float32),
            pltpu.VMEM((ROWS_PER_TILE,), jnp.float32),
            pltpu.VMEM_SHARED((NP,), jnp.float32),
        ],
    )
    def hist(dst_hbm, out_hbm, idx_v, ones_v, zer_v, acc):
        cid = lax.axis_index("c")
        sid = lax.axis_index("s")
        w = cid * NS + sid
        nch_c = jnp.where(cid == 0, nch0, nch1)
        for t in range(CW // 16):
            ones_v[pl.ds(16 * t, 16)] = jnp.ones((16,), jnp.float32)
        for t in range(ROWS_PER_TILE // 16):
            zer_v[pl.ds(16 * t, 16)] = jnp.zeros((16,), jnp.float32)
        pltpu.sync_copy(zer_v, acc.at[pl.ds(sid * ROWS_PER_TILE, ROWS_PER_TILE)])
        plsc.subcore_barrier()
        pltpu.sync_copy(dst_hbm.at[w], idx_v)

        def body(c, _):
            pltpu.sync_copy(ones_v, acc.at[idx_v.at[c]], add=True)
            return 0

        lax.fori_loop(0, nch_c, body, 0)
        plsc.subcore_barrier()
        sl = pl.ds(sid * ROWS_PER_TILE, ROWS_PER_TILE)
        pltpu.sync_copy(acc.at[sl], out_hbm.at[cid].at[sl])

    return hist


def _agg_kernel(nch0, nch1, ph):
    mesh = plsc.VectorSubcoreMesh(core_axis_name="c", subcore_axis_name="s")

    @functools.partial(
        pl.kernel, mesh=mesh,
        out_type=jax.ShapeDtypeStruct((NC, NP, H), jnp.float32),
        scratch_types=[
            pltpu.VMEM((ph, CW), jnp.int32),
            pltpu.VMEM((ph, CW), jnp.int32),
            pltpu.VMEM((CW, H), jnp.float32),
            pltpu.VMEM((CW, H), jnp.float32),
            pltpu.SemaphoreType.DMA,
            pltpu.SemaphoreType.DMA,
            pltpu.VMEM_SHARED((NP, H), jnp.float32),
        ],
    )
    def agg(y_hbm, src_hbm, dsti_hbm, zeros_hbm, out_hbm,
            src_v, dst_v, buf0, buf1, sem0, sem1, acc):
        cid = lax.axis_index("c")
        sid = lax.axis_index("s")
        w = cid * NS + sid
        nph = jnp.where(cid == 0, nch0 // ph, nch1 // ph)
        rsl = pl.ds(sid * ROWS_PER_TILE, ROWS_PER_TILE)
        pltpu.sync_copy(zeros_hbm, acc.at[rsl])
        plsc.subcore_barrier()

        def phase(p, _):
            off = p * ph
            pltpu.sync_copy(src_hbm.at[w].at[pl.ds(off, ph)], src_v)
            pltpu.sync_copy(dsti_hbm.at[w].at[pl.ds(off, ph)], dst_v)
            pltpu.async_copy(y_hbm.at[src_v.at[0]], buf0, sem0)

            def body(i, _):
                c0 = i * 2
                c1 = i * 2 + 1
                pltpu.async_copy(y_hbm.at[src_v.at[c1]], buf1, sem1)
                pltpu.make_async_copy(y_hbm.at[src_v.at[c0]], buf0, sem0).wait()
                pltpu.sync_copy(buf0, acc.at[dst_v.at[c0]], add=True)

                @pl.when(i < ph // 2 - 1)
                def _():
                    pltpu.async_copy(y_hbm.at[src_v.at[c0 + 2]], buf0, sem0)

                pltpu.make_async_copy(y_hbm.at[src_v.at[c1]], buf1, sem1).wait()
                pltpu.sync_copy(buf1, acc.at[dst_v.at[c1]], add=True)
                return 0

            lax.fori_loop(0, ph // 2, body, 0)
            return 0

        lax.fori_loop(0, nph, phase, 0)
        plsc.subcore_barrier()
        pltpu.sync_copy(acc.at[rsl], out_hbm.at[cid].at[rsl])

    return agg


def _graphnorm(X, w, b, ms, ind, indt, cnt):
    mean = jnp.dot(ind, X, preferred_element_type=jnp.float32) / cnt
    cen = X - ms * jnp.dot(indt, mean, preferred_element_type=jnp.float32)
    var = jnp.dot(ind, cen * cen, preferred_element_type=jnp.float32) / cnt
    invstd = lax.rsqrt(var + 1e-5)
    return w * cen * jnp.dot(indt, invstd, preferred_element_type=jnp.float32) + b


def _leaky(v):
    return jnp.where(v >= 0, v, 0.01 * v)


def _prep_body(h_ref, wt_ref, h0_ref, h1_ref, y_ref, dinv_ref):
    dinv = lax.rsqrt(1.0 + h0_ref[...] + h1_ref[...])
    dinv_ref[...] = dinv
    y_ref[...] = jnp.dot(h_ref[...], wt_ref[...],
                         preferred_element_type=jnp.float32) * dinv


def _prep(h, wt, hist0, hist1):
    return pl.pallas_call(
        _prep_body,
        out_shape=(jax.ShapeDtypeStruct((NP, H), jnp.float32),
                   jax.ShapeDtypeStruct((NP, 1), jnp.float32)),
    )(h, wt, hist0, hist1)


def _road_post_body(p0_ref, p1_ref, y_ref, dinv_ref, cb_ref,
                    nw_ref, nb_ref, nms_ref, ind_ref, indt_ref, fwt_ref,
                    h_ref, y2_ref):
    ind = ind_ref[...]
    indt = indt_ref[...]
    cnt = jnp.maximum(jnp.sum(ind, axis=1, keepdims=True), 1.0)
    X = dinv_ref[...] * (p0_ref[...] + p1_ref[...] + y_ref[...]) + cb_ref[...]
    X = _leaky(_graphnorm(X, nw_ref[...], nb_ref[...], nms_ref[...],
                          ind, indt, cnt))
    h_ref[...] = X
    y2_ref[...] = jnp.dot(X, fwt_ref[...], preferred_element_type=jnp.float32)


def _road_post(p0, p1, y, dinv, cb, nw, nb, nms, ind, indt, fwt):
    return pl.pallas_call(
        _road_post_body,
        out_shape=(jax.ShapeDtypeStruct((NP, H), jnp.float32),
                   jax.ShapeDtypeStruct((NP, H), jnp.float32)),
    )(p0, p1, y, dinv, cb, nw, nb, nms, ind, indt, fwt)


def _feat_post_body(q0_ref, q1_ref, y2_ref, hp_ref, fcb_ref,
                    nw_ref, nb_ref, nms_ref, ind_ref, indt_ref,
                    cwt_ref, dinv_ref, h_ref, y_ref):
    ind = ind_ref[...]
    indt = indt_ref[...]
    cnt = jnp.maximum(jnp.sum(ind, axis=1, keepdims=True), 1.0)
    F = 0.25 * (q0_ref[...] + q1_ref[...] + y2_ref[...]) + fcb_ref[...]
    F = _leaky(_graphnorm(F, nw_ref[...], nb_ref[...], nms_ref[...],
                          ind, indt, cnt))
    hn = (hp_ref[...] + F) / 2.0
    h_ref[...] = hn
    y_ref[...] = jnp.dot(hn, cwt_ref[...],
                         preferred_element_type=jnp.float32) * dinv_ref[...]


def _feat_post(q0, q1, y2, hp, fcb, nw, nb, nms, ind, indt, cwt, dinv):
    return pl.pallas_call(
        _feat_post_body,
        out_shape=(jax.ShapeDtypeStruct((NP, H), jnp.float32),
                   jax.ShapeDtypeStruct((NP, H), jnp.float32)),
    )(q0, q1, y2, hp, fcb, nw, nb, nms, ind, indt, cwt, dinv)


def _final_body(q0_ref, q1_ref, y2_ref, hp_ref, fcb_ref,
                nw_ref, nb_ref, nms_ref, ind_ref, indt_ref, h1_ref, gf_ref):
    ind = ind_ref[...]
    indt = indt_ref[...]
    cnt = jnp.maximum(jnp.sum(ind, axis=1, keepdims=True), 1.0)
    F = 0.25 * (q0_ref[...] + q1_ref[...] + y2_ref[...]) + fcb_ref[...]
    F = _leaky(_graphnorm(F, nw_ref[...], nb_ref[...], nms_ref[...],
                          ind, indt, cnt))
    h2 = (hp_ref[...] + F) / 2.0
    p1 = jnp.dot(ind, h1_ref[...], preferred_element_type=jnp.float32) / cnt
    p2 = jnp.dot(ind, h2, preferred_element_type=jnp.float32) / cnt
    gf_ref[...] = p1 + 2.0 * p2


def _final(q0, q1, y2, hp, fcb, nw, nb, nms, ind, indt, h1):
    return pl.pallas_call(
        _final_body,
        out_shape=jax.ShapeDtypeStruct((G, H), jnp.float32),
    )(q0, q1, y2, hp, fcb, nw, nb, nms, ind, indt, h1)


def _pad_edges(src, dst, nch0, nch1):
    cap0 = NS * nch0 * CW
    tot = cap0 + NS * nch1 * CW
    pad = tot - src.shape[0]

    def build(a):
        a_p = jnp.concatenate([a, jnp.full((pad,), NP - 1, jnp.int32)])
        a0 = a_p[:cap0].reshape(NS, nch0, CW)
        a1 = a_p[cap0:].reshape(NS, nch1, CW)
        a1 = jnp.concatenate(
            [a1, jnp.full((NS, nch0 - nch1, CW), NP - 1, jnp.int32)], axis=1)
        return jnp.concatenate([a0, a1], axis=0)

    return build(src), build(dst)


def kernel(x, edge_index, batch, W_emb, b_emb, conv_W, conv_b, norm_w, norm_b,
           norm_ms, fconv_W, fconv_b, fnorm_w, fnorm_b, fnorm_ms):
    pad = NP - N
    x_p = jnp.concatenate([x, jnp.zeros((pad, D), x.dtype)])
    batch_p = jnp.concatenate([batch, jnp.full((pad,), G + 63, jnp.int32)])
    brow = batch_p.reshape(1, NP)
    bcol = batch_p.reshape(NP, 1)
    bmin = batch_p.reshape(NCHUNKS, RB).min(axis=1)
    bmax = batch_p.reshape(NCHUNKS, RB).max(axis=1)

    h0, xn, ind, indt = _embed(x_p, W_emb.T, b_emb.reshape(1, H), brow, bcol)
    nbr = _knn(xn, brow, bcol, bmin, bmax)

    src_p, dst_p = _pad_edges(edge_index[0], edge_index[1], 136, 24)
    fsrc = nbr[:N].reshape(N * K)
    fdst = jnp.repeat(jnp.arange(N, dtype=jnp.int32), K)
    fsrc_p, fdst_p = _pad_edges(fsrc, fdst, 12, 4)

    hist = _hist_kernel(136, 24)(dst_p)
    hist0 = hist[0].reshape(NP, 1)
    hist1 = hist[1].reshape(NP, 1)

    zeros_rows = jnp.zeros((ROWS_PER_TILE, H), jnp.float32)
    agg_r = _agg_kernel(136, 24, 8)
    agg_f = _agg_kernel(12, 4, 4)

    h = h0
    y, dinv = _prep(h, conv_W[0].T, hist0, hist1)
    p = agg_r(y, src_p, dst_p, zeros_rows)
    hp, y2 = _road_post(p[0], p[1], y, dinv, conv_b[0].reshape(1, H),
                        norm_w[0].reshape(1, H), norm_b[0].reshape(1, H),
                        norm_ms[0].reshape(1, H), ind, indt, fconv_W[0].T)
    q = agg_f(y2, fsrc_p, fdst_p, zeros_rows)
    h1, y = _feat_post(q[0], q[1], y2, hp, fconv_b[0].reshape(1, H),
                       fnorm_w[0].reshape(1, H), fnorm_b[0].reshape(1, H),
                       fnorm_ms[0].reshape(1, H), ind, indt,
                       conv_W[1].T, dinv)

    p = agg_r(y, src_p, dst_p, zeros_rows)
    hp, y2 = _road_post(p[0], p[1], y, dinv, conv_b[1].reshape(1, H),
                        norm_w[1].reshape(1, H), norm_b[1].reshape(1, H),
                        norm_ms[1].reshape(1, H), ind, indt, fconv_W[1].T)
    q = agg_f(y2, fsrc_p, fdst_p, zeros_rows)
    gf = _final(q[0], q[1], y2, hp, fconv_b[1].reshape(1, H),
                fnorm_w[1].reshape(1, H), fnorm_b[1].reshape(1, H),
                fnorm_ms[1].reshape(1, H), ind, indt, h1)
    return (gf, 0)


# speedup vs baseline: 1.3927x; 1.0794x over previous
import functools

import jax
import jax.numpy as jnp
from jax import lax
from jax.experimental import pallas as pl
from jax.experimental.pallas import tpu as pltpu
from jax.experimental.pallas import tpu_sc as plsc

N = 10000
D = 128
H = 128
G = 64
K = 3
NUM_LAYERS = 2

NP = 10240
RB = 512
NCHUNKS = NP // RB
NC = 2
NS = 16
NW = NC * NS
ROWS_PER_TILE = NP // NS
CW = 128

NEG = -1e30
IMAX = 2**31 - 1


def _embed_body(x_ref, wt_ref, b_ref, brow_ref, bcol_ref,
                h_ref, xn_ref, ind_ref, indt_ref):
    h = jnp.dot(x_ref[...], wt_ref[...], preferred_element_type=jnp.float32)
    h = h + b_ref[...]
    h_ref[...] = h
    nrm = jnp.sqrt(jnp.sum(h * h, axis=1, keepdims=True))
    xn_ref[...] = h / jnp.maximum(nrm, 1e-12)
    gid_r = lax.broadcasted_iota(jnp.int32, (G, NP), 0)
    ind_ref[...] = (gid_r == brow_ref[...]).astype(jnp.float32)
    gid_c = lax.broadcasted_iota(jnp.int32, (NP, G), 1)
    indt_ref[...] = (gid_c == bcol_ref[...]).astype(jnp.float32)


def _embed(x_p, wembT, b_emb, brow, bcol):
    return pl.pallas_call(
        _embed_body,
        out_shape=(
            jax.ShapeDtypeStruct((NP, H), jnp.float32),
            jax.ShapeDtypeStruct((NP, H), jnp.float32),
            jax.ShapeDtypeStruct((G, NP), jnp.float32),
            jax.ShapeDtypeStruct((NP, G), jnp.float32),
        ),
    )(x_p, wembT, b_emb, brow, bcol)


def _knn_body(bmin_ref, bmax_ref, xn_ref, brow_ref, bcol_ref, nbr_ref,
              rv_ref, ri_ref):
    i = pl.program_id(0)
    rmin = bmin_ref[i]
    rmax = bmax_ref[i]
    xb = xn_ref[pl.ds(pl.multiple_of(i * RB, RB), RB), :]
    bb = bcol_ref[...]
    rv_ref[...] = jnp.full((RB, K), NEG, jnp.float32)
    ri_ref[...] = jnp.zeros((RB, K), jnp.int32)
    rowid = i * RB + lax.broadcasted_iota(jnp.int32, (RB, RB), 0)

    def chunk(j, _):
        cmin = bmin_ref[j]
        cmax = bmax_ref[j]

        @pl.when(jnp.logical_and(cmin <= rmax, cmax >= rmin))
        def _():
            off = pl.multiple_of(j * RB, RB)
            xc = xn_ref[pl.ds(off, RB), :]
            bc = brow_ref[:, pl.ds(off, RB)]
            sim = lax.dot_general(xb, xc, (((1,), (1,)), ((), ())),
                                  preferred_element_type=jnp.float32)
            colid = j * RB + lax.broadcasted_iota(jnp.int32, (RB, RB), 1)
            valid = jnp.logical_and(bb == bc, colid != rowid)
            s = jnp.where(valid, sim, NEG)
            lv, li = [], []
            for _t in range(K):
                m = jnp.max(s, axis=1, keepdims=True)
                sel = jnp.where(s == m, colid, IMAX)
                jm = jnp.min(sel, axis=1, keepdims=True)
                lv.append(m)
                li.append(jm)
                s = jnp.where(colid == jm, NEG, s)
            cand_v = jnp.concatenate([rv_ref[...]] + lv, axis=1)
            cand_i = jnp.concatenate([ri_ref[...]] + li, axis=1)
            nv, ni = [], []
            for _t in range(K):
                m = jnp.max(cand_v, axis=1, keepdims=True)
                sel = jnp.where(cand_v == m, cand_i, IMAX)
                jm = jnp.min(sel, axis=1, keepdims=True)
                nv.append(m)
                ni.append(jm)
                kill = jnp.logical_and(cand_i == jm, cand_v == m)
                cand_v = jnp.where(kill, NEG, cand_v)
                cand_i = jnp.where(kill, IMAX, cand_i)
            rv_ref[...] = jnp.concatenate(nv, axis=1)
            ri_ref[...] = jnp.concatenate(ni, axis=1)
        return 0

    lax.fori_loop(0, NCHUNKS, chunk, 0)
    nbr_ref[...] = jnp.minimum(ri_ref[...], NP - 1)


def _knn(xn, brow, bcol, bmin, bmax):
    return pl.pallas_call(
        _knn_body,
        grid=(NCHUNKS,),
        in_specs=[
            pl.BlockSpec(memory_space=pltpu.SMEM),
            pl.BlockSpec(memory_space=pltpu.SMEM),
            pl.BlockSpec((NP, H), lambda i: (0, 0)),
            pl.BlockSpec((1, NP), lambda i: (0, 0)),
            pl.BlockSpec((RB, 1), lambda i: (i, 0)),
        ],
        out_specs=pl.BlockSpec((RB, K), lambda i: (i, 0)),
        out_shape=jax.ShapeDtypeStruct((NP, K), jnp.int32),
        scratch_shapes=[
            pltpu.VMEM((RB, K), jnp.float32),
            pltpu.VMEM((RB, K), jnp.int32),
        ],
    )(bmin, bmax, xn, brow, bcol)


def _hist_kernel(nch0, nch1):
    mesh = plsc.VectorSubcoreMesh(core_axis_name="c", subcore_axis_name="s")

    @functools.partial(
        pl.kernel, mesh=mesh,
        out_type=jax.ShapeDtypeStruct((NC, NP), jnp.float32),
        scratch_types=[
            pltpu.VMEM((nch0, CW), jnp.int32),
            pltpu.VMEM((CW,), jnp.float32),
            pltpu.VMEM((ROWS_PER_TILE,), jnp.float32),
            pltpu.VMEM_SHARED((NP,), jnp.float32),
        ],
    )
    def hist(dst_hbm, out_hbm, idx_v, ones_v, zer_v, acc):
        cid = lax.axis_index("c")
        sid = lax.axis_index("s")
        w = cid * NS + sid
        nch_c = jnp.where(cid == 0, nch0, nch1)
        for t in range(CW // 16):
            ones_v[pl.ds(16 * t, 16)] = jnp.ones((16,), jnp.float32)
        for t in range(ROWS_PER_TILE // 16):
            zer_v[pl.ds(16 * t, 16)] = jnp.zeros((16,), jnp.float32)
        pltpu.sync_copy(zer_v, acc.at[pl.ds(sid * ROWS_PER_TILE, ROWS_PER_TILE)])
        plsc.subcore_barrier()
        pltpu.sync_copy(dst_hbm.at[w], idx_v)

        def body(c, _):
            pltpu.sync_copy(ones_v, acc.at[idx_v.at[c]], add=True)
            return 0

        lax.fori_loop(0, nch_c, body, 0)
        plsc.subcore_barrier()
        sl = pl.ds(sid * ROWS_PER_TILE, ROWS_PER_TILE)
        pltpu.sync_copy(acc.at[sl], out_hbm.at[cid].at[sl])

    return hist


def _agg_kernel(nch0, nch1, ph):
    mesh = plsc.VectorSubcoreMesh(core_axis_name="c", subcore_axis_name="s")

    @functools.partial(
        pl.kernel, mesh=mesh,
        out_type=jax.ShapeDtypeStruct((NC, NP, H), jnp.float32),
        scratch_types=[
            pltpu.VMEM((ph, CW), jnp.int32),
            pltpu.VMEM((ph, CW), jnp.int32),
            pltpu.VMEM((CW, H), jnp.float32),
            pltpu.VMEM((CW, H), jnp.float32),
            pltpu.SemaphoreType.DMA,
            pltpu.SemaphoreType.DMA,
            pltpu.VMEM_SHARED((NP, H), jnp.float32),
        ],
    )
    def agg(y_hbm, src_hbm, dsti_hbm, zeros_hbm, out_hbm,
            src_v, dst_v, buf0, buf1, sem0, sem1, acc):
        cid = lax.axis_index("c")
        sid = lax.axis_index("s")
        w = cid * NS + sid
        nph = jnp.where(cid == 0, nch0 // ph, nch1 // ph)
        rsl = pl.ds(sid * ROWS_PER_TILE, ROWS_PER_TILE)
        pltpu.sync_copy(zeros_hbm, acc.at[rsl])
        plsc.subcore_barrier()

        def phase(p, _):
            off = p * ph
            pltpu.sync_copy(src_hbm.at[w].at[pl.ds(off, ph)], src_v)
            pltpu.sync_copy(dsti_hbm.at[w].at[pl.ds(off, ph)], dst_v)
            pltpu.async_copy(y_hbm.at[src_v.at[0]], buf0, sem0)

            def body(i, _):
                c0 = i * 2
                c1 = i * 2 + 1
                pltpu.async_copy(y_hbm.at[src_v.at[c1]], buf1, sem1)
                pltpu.make_async_copy(y_hbm.at[src_v.at[c0]], buf0, sem0).wait()
                pltpu.sync_copy(buf0, acc.at[dst_v.at[c0]], add=True)

                @pl.when(i < ph // 2 - 1)
                def _():
                    pltpu.async_copy(y_hbm.at[src_v.at[c0 + 2]], buf0, sem0)

                pltpu.make_async_copy(y_hbm.at[src_v.at[c1]], buf1, sem1).wait()
                pltpu.sync_copy(buf1, acc.at[dst_v.at[c1]], add=True)
                return 0

            lax.fori_loop(0, ph // 2, body, 0)
            return 0

        lax.fori_loop(0, nph, phase, 0)
        plsc.subcore_barrier()
        pltpu.sync_copy(acc.at[rsl], out_hbm.at[cid].at[rsl])

    return agg


def _graphnorm(X, w, b, ms, ind, indt, cnt):
    mean = jnp.dot(ind, X, preferred_element_type=jnp.float32) / cnt
    cen = X - ms * jnp.dot(indt, mean, preferred_element_type=jnp.float32)
    var = jnp.dot(ind, cen * cen, preferred_element_type=jnp.float32) / cnt
    invstd = lax.rsqrt(var + 1e-5)
    return w * cen * jnp.dot(indt, invstd, preferred_element_type=jnp.float32) + b


def _leaky(v):
    return jnp.where(v >= 0, v, 0.01 * v)


def _prep_body(h_ref, wt_ref, h0_ref, h1_ref, y_ref, dinv_ref):
    dinv = lax.rsqrt(1.0 + h0_ref[...] + h1_ref[...])
    dinv_ref[...] = dinv
    y_ref[...] = jnp.dot(h_ref[...], wt_ref[...],
                         preferred_element_type=jnp.float32) * dinv


def _prep(h, wt, hist0, hist1):
    return pl.pallas_call(
        _prep_body,
        out_shape=(jax.ShapeDtypeStruct((NP, H), jnp.float32),
                   jax.ShapeDtypeStruct((NP, 1), jnp.float32)),
    )(h, wt, hist0, hist1)


def _road_post_body(p0_ref, p1_ref, y_ref, dinv_ref, cb_ref,
                    nw_ref, nb_ref, nms_ref, ind_ref, indt_ref, fwt_ref,
                    h_ref, y2_ref):
    ind = ind_ref[...]
    indt = indt_ref[...]
    cnt = jnp.maximum(jnp.sum(ind, axis=1, keepdims=True), 1.0)
    X = dinv_ref[...] * (p0_ref[...] + p1_ref[...] + y_ref[...]) + cb_ref[...]
    X = _leaky(_graphnorm(X, nw_ref[...], nb_ref[...], nms_ref[...],
                          ind, indt, cnt))
    h_ref[...] = X
    y2_ref[...] = jnp.dot(X, fwt_ref[...], preferred_element_type=jnp.float32)


def _road_post(p0, p1, y, dinv, cb, nw, nb, nms, ind, indt, fwt):
    return pl.pallas_call(
        _road_post_body,
        out_shape=(jax.ShapeDtypeStruct((NP, H), jnp.float32),
                   jax.ShapeDtypeStruct((NP, H), jnp.float32)),
    )(p0, p1, y, dinv, cb, nw, nb, nms, ind, indt, fwt)


def _feat_post_body(q0_ref, q1_ref, y2_ref, hp_ref, fcb_ref,
                    nw_ref, nb_ref, nms_ref, ind_ref, indt_ref,
                    cwt_ref, dinv_ref, h_ref, y_ref):
    ind = ind_ref[...]
    indt = indt_ref[...]
    cnt = jnp.maximum(jnp.sum(ind, axis=1, keepdims=True), 1.0)
    F = 0.25 * (q0_ref[...] + q1_ref[...] + y2_ref[...]) + fcb_ref[...]
    F = _leaky(_graphnorm(F, nw_ref[...], nb_ref[...], nms_ref[...],
                          ind, indt, cnt))
    hn = (hp_ref[...] + F) / 2.0
    h_ref[...] = hn
    y_ref[...] = jnp.dot(hn, cwt_ref[...],
                         preferred_element_type=jnp.float32) * dinv_ref[...]


def _feat_post(q0, q1, y2, hp, fcb, nw, nb, nms, ind, indt, cwt, dinv):
    return pl.pallas_call(
        _feat_post_body,
        out_shape=(jax.ShapeDtypeStruct((NP, H), jnp.float32),
                   jax.ShapeDtypeStruct((NP, H), jnp.float32)),
    )(q0, q1, y2, hp, fcb, nw, nb, nms, ind, indt, cwt, dinv)


def _final_body(q0_ref, q1_ref, y2_ref, hp_ref, fcb_ref,
                nw_ref, nb_ref, nms_ref, ind_ref, indt_ref, h1_ref, gf_ref):
    ind = ind_ref[...]
    indt = indt_ref[...]
    cnt = jnp.maximum(jnp.sum(ind, axis=1, keepdims=True), 1.0)
    F = 0.25 * (q0_ref[...] + q1_ref[...] + y2_ref[...]) + fcb_ref[...]
    F = _leaky(_graphnorm(F, nw_ref[...], nb_ref[...], nms_ref[...],
                          ind, indt, cnt))
    h2 = (hp_ref[...] + F) / 2.0
    p1 = jnp.dot(ind, h1_ref[...], preferred_element_type=jnp.float32) / cnt
    p2 = jnp.dot(ind, h2, preferred_element_type=jnp.float32) / cnt
    gf_ref[...] = p1 + 2.0 * p2


def _final(q0, q1, y2, hp, fcb, nw, nb, nms, ind, indt, h1):
    return pl.pallas_call(
        _final_body,
        out_shape=jax.ShapeDtypeStruct((G, H), jnp.float32),
    )(q0, q1, y2, hp, fcb, nw, nb, nms, ind, indt, h1)


def _pad_edges(src, dst, nch0, nch1):
    cap0 = NS * nch0 * CW
    tot = cap0 + NS * nch1 * CW
    pad = tot - src.shape[0]

    def build(a):
        a_p = jnp.concatenate([a, jnp.full((pad,), NP - 1, jnp.int32)])
        a0 = a_p[:cap0].reshape(NS, nch0, CW)
        a1 = a_p[cap0:].reshape(NS, nch1, CW)
        a1 = jnp.concatenate(
            [a1, jnp.full((NS, nch0 - nch1, CW), NP - 1, jnp.int32)], axis=1)
        return jnp.concatenate([a0, a1], axis=0)

    return build(src), build(dst)


def kernel(x, edge_index, batch, W_emb, b_emb, conv_W, conv_b, norm_w, norm_b,
           norm_ms, fconv_W, fconv_b, fnorm_w, fnorm_b, fnorm_ms):
    pad = NP - N
    x_p = jnp.concatenate([x, jnp.zeros((pad, D), x.dtype)])
    batch_p = jnp.concatenate([batch, jnp.full((pad,), G + 63, jnp.int32)])
    brow = batch_p.reshape(1, NP)
    bcol = batch_p.reshape(NP, 1)
    bmin = batch_p.reshape(NCHUNKS, RB).min(axis=1)
    bmax = batch_p.reshape(NCHUNKS, RB).max(axis=1)

    h0, xn, ind, indt = _embed(x_p, W_emb.T, b_emb.reshape(1, H), brow, bcol)
    nbr = _knn(xn, brow, bcol, bmin, bmax)

    src_p, dst_p = _pad_edges(edge_index[0], edge_index[1], 144, 16)
    fsrc = nbr[:N].reshape(N * K)
    fdst = jnp.repeat(jnp.arange(N, dtype=jnp.int32), K)
    fsrc_p, fdst_p = _pad_edges(fsrc, fdst, 12, 4)

    hist = _hist_kernel(144, 16)(dst_p)
    hist0 = hist[0].reshape(NP, 1)
    hist1 = hist[1].reshape(NP, 1)

    zeros_rows = jnp.zeros((ROWS_PER_TILE, H), jnp.float32)
    agg_r = _agg_kernel(144, 16, 8)
    agg_f = _agg_kernel(12, 4, 4)

    h = h0
    y, dinv = _prep(h, conv_W[0].T, hist0, hist1)
    p = agg_r(y, src_p, dst_p, zeros_rows)
    hp, y2 = _road_post(p[0], p[1], y, dinv, conv_b[0].reshape(1, H),
                        norm_w[0].reshape(1, H), norm_b[0].reshape(1, H),
                        norm_ms[0].reshape(1, H), ind, indt, fconv_W[0].T)
    q = agg_f(y2, fsrc_p, fdst_p, zeros_rows)
    h1, y = _feat_post(q[0], q[1], y2, hp, fconv_b[0].reshape(1, H),
                       fnorm_w[0].reshape(1, H), fnorm_b[0].reshape(1, H),
                       fnorm_ms[0].reshape(1, H), ind, indt,
                       conv_W[1].T, dinv)

    p = agg_r(y, src_p, dst_p, zeros_rows)
    hp, y2 = _road_post(p[0], p[1], y, dinv, conv_b[1].reshape(1, H),
                        norm_w[1].reshape(1, H), norm_b[1].reshape(1, H),
                        norm_ms[1].reshape(1, H), ind, indt, fconv_W[1].T)
    q = agg_f(y2, fsrc_p, fdst_p, zeros_rows)
    gf = _final(q[0], q[1], y2, hp, fconv_b[1].reshape(1, H),
                fnorm_w[1].reshape(1, H), fnorm_b[1].reshape(1, H),
                fnorm_ms[1].reshape(1, H), ind, indt, h1)
    return (gf, 0)
